# Initial kernel scaffold; baseline (speedup 1.0000x reference)
import jax
import jax.numpy as jnp
from jax.experimental import pallas as pl


def kernel(doc_ids, context_ids, sample_ids, paragraph_matrix, word_matrix, outputs):
    raise NotImplementedError("write your pallas kernel here")


# speedup vs baseline: 2.2646x; 2.2646x over previous
import functools

import jax
import jax.numpy as jnp
from jax import lax
from jax.experimental import pallas as pl
from jax.experimental.pallas import tpu as pltpu
from jax.experimental.pallas import tpu_sc as plsc

D = 64
B = 4096
CTX = 20
SAMP = 20
NC, NSUB = 2, 16
NW = NC * NSUB
BPW = B // NW
NB = 16
NBLK = BPW // NB
ROWS = NB * CTX
VL = 16


def _dot_partial(acc, sr, r):
    p = acc[0] * sr[r, pl.ds(0, VL)]
    for k in range(1, 4):
        p = p + acc[k] * sr[r, pl.ds(k * VL, VL)]
    return p


def _sc_body(doc_hbm, cidx_hbm, sidx_hbm, para_hbm, word_hbm, outt_hbm,
             out_hbm, didx, cidx, sidx, drows, crows, srows, lg, pbuf,
             gsem0, gsem1, osem0, osem1):
    wid = lax.axis_index("s") * NC + lax.axis_index("c")
    wbase = wid * BPW

    pltpu.sync_copy(doc_hbm.at[pl.ds(wbase, BPW)], didx)
    pltpu.sync_copy(cidx_hbm.at[pl.ds(wbase * CTX, BPW * CTX)], cidx)
    pltpu.sync_copy(sidx_hbm.at[pl.ds(wbase * SAMP, BPW * SAMP)], sidx)

    gsems = (gsem0, gsem1)
    osems = (osem0, osem1)

    def fire(j, slot):
        base = j * NB
        hs = [pltpu.async_copy(
            para_hbm.at[didx.at[pl.ds(base, NB)]], drows.at[slot],
            gsems[slot])]
        for off, sz in ((0, 128), (128, 128), (256, 64)):
            hs.append(pltpu.async_copy(
                word_hbm.at[cidx.at[pl.ds(base * CTX + off, sz)]],
                crows.at[slot].at[pl.ds(off, sz)], gsems[slot]))
            hs.append(pltpu.async_copy(
                outt_hbm.at[sidx.at[pl.ds(base * SAMP + off, sz)]],
                srows.at[slot].at[pl.ds(off, sz)], gsems[slot]))
        return hs

    def compute(slot):
        dr = drows.at[slot]
        cr = crows.at[slot]
        sr = srows.at[slot]
        lgs = lg.at[slot]
        lane16 = lax.iota(jnp.int32, VL) * VL

        def body(b, carry):
            acc = [dr[b, pl.ds(k * VL, VL)] for k in range(4)]
            for c in range(CTX):
                r = b * CTX + c
                for k in range(4):
                    acc[k] = acc[k] + cr[r, pl.ds(k * VL, VL)]
            for s in range(SAMP):
                pbuf[pl.ds(s * VL, VL)] = _dot_partial(acc, sr, b * SAMP + s)
            for g in range(2):
                r = plsc.load_gather(pbuf, [lane16 + g * VL * VL])
                for k in range(1, VL):
                    r = r + plsc.load_gather(pbuf, [lane16 + (g * VL * VL + k)])
                lgs[b, pl.ds(g * VL, VL)] = r
            return carry

        lax.fori_loop(0, NB, body, 0)

    handles = [None, None]
    out_handles = [None, None]
    handles[0] = fire(0, 0)
    for j in range(NBLK):
        slot = j & 1
        if j + 1 < NBLK:
            handles[1 - slot] = fire(j + 1, 1 - slot)
        for h in handles[slot]:
            h.wait()
        if out_handles[slot] is not None:
            out_handles[slot].wait()
        compute(slot)
        out_handles[slot] = pltpu.async_copy(
            lg.at[slot], out_hbm.at[pl.ds(wbase + j * NB, NB)], osems[slot])
    for oh in out_handles:
        if oh is not None:
            oh.wait()


_sc_kernel = functools.partial(
    pl.kernel,
    out_type=jax.ShapeDtypeStruct((B, 2 * VL), jnp.float32),
    mesh=plsc.VectorSubcoreMesh(core_axis_name="c", subcore_axis_name="s"),
    compiler_params=pltpu.CompilerParams(
        needs_layout_passes=False, use_tc_tiling_on_sc=False),
    scratch_types=[
        pltpu.VMEM((BPW,), jnp.int32),
        pltpu.VMEM((BPW * CTX,), jnp.int32),
        pltpu.VMEM((BPW * SAMP,), jnp.int32),
        pltpu.VMEM((2, NB, D), jnp.float32),
        pltpu.VMEM((2, ROWS, D), jnp.float32),
        pltpu.VMEM((2, ROWS, D), jnp.float32),
        pltpu.VMEM((2, NB, 2 * VL), jnp.float32),
        pltpu.VMEM((2 * VL * VL,), jnp.float32),
        pltpu.SemaphoreType.DMA,
        pltpu.SemaphoreType.DMA,
        pltpu.SemaphoreType.DMA,
        pltpu.SemaphoreType.DMA,
    ],
)(_sc_body)


def kernel(doc_ids, context_ids, sample_ids, paragraph_matrix, word_matrix,
           outputs):
    doc_i = doc_ids.astype(jnp.int32)
    ctx_i = context_ids.astype(jnp.int32).reshape(-1)
    samp_i = sample_ids.astype(jnp.int32).reshape(-1)
    outt = outputs.T
    padded = _sc_kernel(doc_i, ctx_i, samp_i, paragraph_matrix,
                        word_matrix, outt)
    return padded[:, :SAMP]
